# baseline (speedup 1.0000x reference)
import functools

import jax
import jax.numpy as jnp
from jax import lax
from jax.experimental import pallas as pl
from jax.experimental.pallas import tpu as pltpu
from jax.experimental.pallas import tpu_sc as plsc

_D = 64
_DP = 128
_SCALE = float(_D) ** 0.5
_G1 = 128
_NBUF = 2


@functools.cache
def _build(batch, seq, vocab):
    info = plsc.get_sparse_core_info()
    nc, ns, nl = info.num_cores, info.num_subcores, info.num_lanes
    nw = nc * ns
    n_idx = batch * seq
    assert batch % nw == 0 and seq % 8 == 0 and _G1 % 8 == 0 and seq > _G1
    g2 = seq - _G1
    assert g2 <= 128 and g2 % 8 == 0
    batches_per_w = batch // nw
    b_per_w = n_idx // nw
    assert batches_per_w >= 2 * _NBUF and batches_per_w % _NBUF == 0

    mesh = plsc.VectorSubcoreMesh(core_axis_name="c", subcore_axis_name="s")

    @functools.partial(
        pl.kernel,
        mesh=mesh,
        out_type=jax.ShapeDtypeStruct((batch, seq, _D), jnp.float32),
        scratch_types=[
            pltpu.VMEM((b_per_w,), jnp.int32),
            pltpu.VMEM((_NBUF, seq, _DP), jnp.float32),
            pltpu.VMEM((_NBUF, seq, _D), jnp.float32),
        ]
        + [pltpu.SemaphoreType.DMA] * (2 * _NBUF),
    )
    def emb_kernel(idx_hbm, table_hbm, out_hbm, idx_v, in_bufs, out_bufs, *sems):
        sin, sout = sems[:_NBUF], sems[_NBUF:]
        wid = lax.axis_index("s") * nc + lax.axis_index("c")
        base = wid * b_per_w
        base_batch = wid * batches_per_w
        pltpu.sync_copy(idx_hbm.at[pl.ds(base, b_per_w)], idx_v)

        def gather_copies(i, b):
            off = i * seq
            c1 = pltpu.make_async_copy(
                table_hbm.at[idx_v.at[pl.ds(off, _G1)]],
                in_bufs.at[b].at[pl.ds(0, _G1)],
                sin[b],
            )
            c2 = pltpu.make_async_copy(
                table_hbm.at[idx_v.at[pl.ds(off + _G1, g2)]],
                in_bufs.at[b].at[pl.ds(_G1, g2)],
                sin[b],
            )
            return c1, c2

        def gather_start(i, b):
            c1, c2 = gather_copies(i, b)
            c1.start()
            c2.start()

        def gather_wait(i, b):
            c1, c2 = gather_copies(i, b)
            c1.wait()
            c2.wait()

        def out_copy(i, b):
            return pltpu.make_async_copy(
                out_bufs.at[b], out_hbm.at[base_batch + i], sout[b]
            )

        def scale(b):
            def body(r4, c):
                for dr in range(4):
                    r = r4 * 4 + dr
                    for j in range(_D // nl):
                        sl = pl.ds(j * nl, nl)
                        out_bufs[b, r, sl] = in_bufs[b, r, sl] * _SCALE
                return c

            lax.fori_loop(0, seq // 4, body, 0)

        for b in range(_NBUF):
            gather_start(b, b)

        for i in range(_NBUF):
            b = i
            gather_wait(i, b)
            scale(b)
            out_copy(i, b).start()
            gather_start(i + _NBUF, b)

        def mid(k, c):
            for b in range(_NBUF):
                i = k * _NBUF + _NBUF + b
                gather_wait(i, b)
                out_copy(i - _NBUF, b).wait()
                scale(b)
                out_copy(i, b).start()
                gather_start(i + _NBUF, b)
            return c

        lax.fori_loop(0, (batches_per_w - 2 * _NBUF) // _NBUF, mid, 0)

        for k in range(_NBUF):
            i = batches_per_w - _NBUF + k
            gather_wait(i, k)
            out_copy(i - _NBUF, k).wait()
            scale(k)
            out_copy(i, k).start()
        for k in range(_NBUF):
            out_copy(batches_per_w - _NBUF + k, k).wait()

    return emb_kernel


def kernel(x, table):
    b, s = x.shape
    v, _ = table.shape
    idx = x.reshape(b * s).astype(jnp.int32)
    padded = jnp.pad(table, ((0, 0), (0, _DP - _D)))
    return _build(b, s, v)(idx, padded)


# speedup vs baseline: 1.1070x; 1.1070x over previous
import functools

import jax
import jax.numpy as jnp
from jax import lax
from jax.experimental import pallas as pl
from jax.experimental.pallas import tpu as pltpu
from jax.experimental.pallas import tpu_sc as plsc

_D = 64
_DP = 128
_SCALE = float(_D) ** 0.5
_CHUNK = 128
_NBUF = 4
_DBLK = 128
_DNBUF = 2


def _mesh():
    return plsc.VectorSubcoreMesh(core_axis_name="c", subcore_axis_name="s")


@functools.cache
def _build_detile(vocab):
    info = plsc.get_sparse_core_info()
    nc, ns, nl = info.num_cores, info.num_subcores, info.num_lanes
    nw = nc * ns
    n_full = vocab // _DBLK
    tail_rows = vocab - n_full * _DBLK
    assert tail_rows % 8 == 0
    nb_lo = n_full // nw
    n_extra = n_full - nb_lo * nw

    @functools.partial(
        pl.kernel,
        mesh=_mesh(),
        out_type=jax.ShapeDtypeStruct((vocab, _DP), jnp.float32),
        scratch_types=[
            pltpu.VMEM((_DNBUF, _DBLK, _D), jnp.float32),
            pltpu.VMEM((_DNBUF, _DBLK, _DP), jnp.float32),
        ]
        + [pltpu.SemaphoreType.DMA] * (2 * _DNBUF),
    )
    def detile_kernel(tab_hbm, pad_hbm, vin, vout, *sems):
        sin, sout = sems[:_DNBUF], sems[_DNBUF:]
        wid = lax.axis_index("s") * nc + lax.axis_index("c")
        nb_w = nb_lo + jnp.where(wid < n_extra, 1, 0)

        def in_copy(k, b):
            g = wid + k * nw
            return pltpu.make_async_copy(
                tab_hbm.at[pl.ds(g * _DBLK, _DBLK)], vin.at[b], sin[b]
            )

        def out_copy(k, b):
            g = wid + k * nw
            return pltpu.make_async_copy(
                vout.at[b], pad_hbm.at[pl.ds(g * _DBLK, _DBLK)], sout[b]
            )

        def repack(b):
            def body(r4, c):
                for dr in range(4):
                    r = r4 * 4 + dr
                    for j in range(_D // nl):
                        sl = pl.ds(j * nl, nl)
                        vout[b, r, sl] = vin[b, r, sl] * _SCALE
                return c

            lax.fori_loop(0, _DBLK // 4, body, 0)

        def step(k, c):
            b0 = lax.rem(k, _DNBUF)
            for b in range(_DNBUF):
                @pl.when(b0 == b)
                def _():
                    in_copy(k, b).wait()
                    @pl.when(k >= _DNBUF)
                    def _():
                        out_copy(k - _DNBUF, b).wait()
                    repack(b)
                    out_copy(k, b).start()
                    @pl.when(k + _DNBUF < nb_w)
                    def _():
                        in_copy(k + _DNBUF, b).start()
            return c

        for b in range(_DNBUF):
            @pl.when(b < nb_w)
            def _():
                in_copy(b, b).start()

        lax.fori_loop(0, nb_w, step, 0)

        def drain(k, c):
            b0 = lax.rem(k, _DNBUF)
            for b in range(_DNBUF):
                @pl.when(b0 == b)
                def _():
                    out_copy(k, b).wait()
            return c

        lax.fori_loop(
            lax.max(nb_w - _DNBUF, 0), nb_w, drain, 0
        )

        if tail_rows:
            @pl.when(wid == 0)
            def _():
                pltpu.sync_copy(
                    tab_hbm.at[pl.ds(n_full * _DBLK, tail_rows)],
                    vin.at[0].at[pl.ds(0, tail_rows)],
                )

                def tbody(r4, c):
                    for dr in range(4):
                        r = r4 * 4 + dr
                        for j in range(_D // nl):
                            sl = pl.ds(j * nl, nl)
                            vout[0, r, sl] = vin[0, r, sl] * _SCALE
                    return c

                lax.fori_loop(0, tail_rows // 4, tbody, 0)
                pltpu.sync_copy(
                    vout.at[0].at[pl.ds(0, tail_rows)],
                    pad_hbm.at[pl.ds(n_full * _DBLK, tail_rows)],
                )

    return detile_kernel


@functools.cache
def _build_gather(n_idx, vocab):
    info = plsc.get_sparse_core_info()
    nc, ns, nl = info.num_cores, info.num_subcores, info.num_lanes
    nw = nc * ns
    assert n_idx % (nw * _CHUNK) == 0
    b_per_w = n_idx // nw
    n_chunks = b_per_w // _CHUNK
    assert n_chunks % _NBUF == 0 and n_chunks >= 2 * _NBUF

    @functools.partial(
        pl.kernel,
        mesh=_mesh(),
        out_type=jax.ShapeDtypeStruct((n_idx, _D), jnp.float32),
        scratch_types=[
            pltpu.VMEM((b_per_w,), jnp.int32),
            pltpu.VMEM((_NBUF, _CHUNK, _DP), jnp.float32),
            pltpu.VMEM((2, _CHUNK, _D), jnp.float32),
        ]
        + [pltpu.SemaphoreType.DMA] * (_NBUF + 2),
    )
    def gather_kernel(idx_hbm, pad_hbm, out_hbm, idx_v, in_bufs, out_bufs, *sems):
        sin, sout = sems[:_NBUF], sems[_NBUF:]
        wid = lax.axis_index("s") * nc + lax.axis_index("c")
        base = wid * b_per_w
        pltpu.sync_copy(idx_hbm.at[pl.ds(base, b_per_w)], idx_v)

        def gather_copy(g, b):
            return pltpu.make_async_copy(
                pad_hbm.at[idx_v.at[pl.ds(g * _CHUNK, _CHUNK)]],
                in_bufs.at[b],
                sin[b],
            )

        def out_copy(g, ob):
            return pltpu.make_async_copy(
                out_bufs.at[ob],
                out_hbm.at[pl.ds(base + g * _CHUNK, _CHUNK)],
                sout[ob],
            )

        def compact(b, ob):
            def body(r4, c):
                for dr in range(4):
                    r = r4 * 4 + dr
                    for j in range(_D // nl):
                        sl = pl.ds(j * nl, nl)
                        out_bufs[ob, r, sl] = in_bufs[b, r, sl]
                return c

            lax.fori_loop(0, _CHUNK // 4, body, 0)

        half = _NBUF // 2

        for g in range(half):
            gather_copy(g, g).start()

        for g in range(half):
            b, ob = g, g % 2
            gather_copy(g, b).wait()
            compact(b, ob)
            out_copy(g, ob).start()
            gather_copy(g + half, (g + half) % _NBUF).start()

        def mid(i, c):
            for k in range(_NBUF):
                g = i * _NBUF + half + k
                b = (half + k) % _NBUF
                ob = (half + k) % 2
                gather_copy(g, b).wait()
                out_copy(g - 2, ob).wait()
                compact(b, ob)
                out_copy(g, ob).start()
                gather_copy(g + half, (b + half) % _NBUF).start()
            return c

        lax.fori_loop(0, (n_chunks - _NBUF) // _NBUF, mid, 0)

        for k in range(half):
            g = n_chunks - half + k
            b, ob = g % _NBUF, g % 2
            gather_copy(g, b).wait()
            out_copy(g - 2, ob).wait()
            compact(b, ob)
            out_copy(g, ob).start()
        for k in range(half):
            g = n_chunks - half + k
            out_copy(g, g % 2).wait()

    return gather_kernel


def kernel(x, table):
    b, s = x.shape
    v, _ = table.shape
    idx = x.reshape(b * s).astype(jnp.int32)
    padded = jnp.pad(table, ((0, 0), (0, _DP - _D))) * _SCALE
    out = _build_gather(b * s, v)(idx, padded)
    return out.reshape(b, s, _D)
